# baseline (speedup 1.0000x reference)
import jax
import jax.numpy as jnp
from jax import lax
from jax.experimental import pallas as pl
from jax.experimental.pallas import tpu as pltpu
from jax.experimental.pallas import tpu_sc as plsc

N_NODES = 100000
B = 625000
D = 128
LANES = 16

NC = 2
NS = 16
NW = NC * NS

C = 120
NBUF = 4
PIPE = 2

GROUPS = B // 8
GPW_BASE = GROUPS // NW
GPW_REM = GROUPS - GPW_BASE * NW
N_MAX = 8 * (GPW_BASE + 1)
N_CHUNKS = NBUF * (-(-(-(-N_MAX // C)) // NBUF))
N_OUTER = N_CHUNKS // NBUF


def _sc_body(idx_hbm, feat_hbm, mem_hbm, out_hbm, *scratch):
    idx_v = scratch[0:NBUF]
    rows_v = scratch[NBUF:2 * NBUF]
    feat_v = scratch[2 * NBUF:3 * NBUF]
    sem_idx = scratch[3 * NBUF:4 * NBUF]
    sem_gat = scratch[4 * NBUF:5 * NBUF]
    sem_fea = scratch[5 * NBUF:6 * NBUF]
    sem_out = scratch[6 * NBUF:7 * NBUF]

    wid = lax.axis_index("s") * NC + lax.axis_index("c")
    extra = jnp.minimum(wid, GPW_REM)
    start = 8 * (wid * GPW_BASE + extra)
    n_rows = 8 * (GPW_BASE + jnp.where(wid < GPW_REM, 1, 0))

    def base(c):
        return start + jnp.minimum(c * C, n_rows - C)

    def fire_idx(c, b):
        pltpu.async_copy(idx_hbm.at[pl.ds(base(c), C)], idx_v[b], sem_idx[b])

    def fire_fetch(c, b):
        pltpu.async_copy(mem_hbm.at[idx_v[b]], rows_v[b], sem_gat[b])
        pltpu.async_copy(feat_hbm.at[pl.ds(base(c), C), :], feat_v[b],
                         sem_fea[b])

    def wait(src, dst, sem):
        pltpu.make_async_copy(src, dst, sem).wait()

    for c in range(PIPE + 1):
        fire_idx(c, c)
    for c in range(PIPE):
        wait(idx_hbm.at[pl.ds(base(c), C)], idx_v[c], sem_idx[c])
        fire_fetch(c, c)

    def outer(g, _):
        for b in range(NBUF):
            c = g * NBUF + b

            @pl.when(c + PIPE + 1 < N_CHUNKS)
            def _():
                fire_idx(c + PIPE + 1, (b + PIPE + 1) % NBUF)

            @pl.when(c + PIPE < N_CHUNKS)
            def _():
                bn = (b + PIPE) % NBUF
                wait(idx_hbm.at[pl.ds(base(c + PIPE), C)], idx_v[bn],
                     sem_idx[bn])

                @pl.when(c >= PIPE)
                def _():
                    wait(rows_v[bn], out_hbm.at[pl.ds(base(c - PIPE), C), :],
                         sem_out[bn])

                fire_fetch(c + PIPE, bn)

            wait(mem_hbm.at[idx_v[b]], rows_v[b], sem_gat[b])
            wait(feat_hbm.at[pl.ds(base(c), C), :], feat_v[b], sem_fea[b])

            def add_row(j, _):
                for k in range(D // LANES):
                    sl = pl.ds(k * LANES, LANES)
                    rows_v[b][j, sl] = rows_v[b][j, sl] + feat_v[b][j, sl]
                return 0

            lax.fori_loop(0, C, add_row, 0)
            pltpu.async_copy(rows_v[b], out_hbm.at[pl.ds(base(c), C), :],
                             sem_out[b])
        return 0

    lax.fori_loop(0, N_OUTER, outer, 0)

    for b in range(NBUF):
        c = N_CHUNKS - NBUF + b
        wait(rows_v[c % NBUF], out_hbm.at[pl.ds(base(c), C), :],
             sem_out[c % NBUF])


@jax.jit
def _gather_add(source_nodes, features, memory):
    mesh = plsc.VectorSubcoreMesh(core_axis_name="c", subcore_axis_name="s")
    f = pl.kernel(
        _sc_body,
        out_type=jax.ShapeDtypeStruct((B, D), jnp.float32),
        mesh=mesh,
        scratch_types=(
            [pltpu.VMEM((C,), jnp.int32) for _ in range(NBUF)]
            + [pltpu.VMEM((C, D), jnp.float32) for _ in range(NBUF)]
            + [pltpu.VMEM((C, D), jnp.float32) for _ in range(NBUF)]
            + [pltpu.SemaphoreType.DMA for _ in range(4 * NBUF)]
        ),
    )
    return f(source_nodes, features, memory)


def kernel(source_nodes, source_node_raw_features, timestamps, n_layers,
           memory, time_W, time_b):
    idx = source_nodes.astype(jnp.int32)
    return _gather_add(idx, source_node_raw_features, memory)
